# Initial kernel scaffold; baseline (speedup 1.0000x reference)
import jax
import jax.numpy as jnp
from jax.experimental import pallas as pl


def kernel(x_offset, x_depth, batch, edge_index, params):
    raise NotImplementedError("write your pallas kernel here")


# speedup vs baseline: 11.0598x; 11.0598x over previous
import functools

import jax
import jax.numpy as jnp
import numpy as np
from jax import lax
from jax.experimental import pallas as pl
from jax.experimental.pallas import tpu as pltpu
from jax.experimental.pallas import tpu_sc as plsc

N = 100000
E = 1600000
B = 64
HID = 32
OUT = 8

NP = 102400
RB = 1600
CH = 128
NCHUNK = E // CH
NW = 32
CPW = NCHUNK // NW
EXTRA = NCHUNK - CPW * NW
TPR = NP // 16
STG = 400

INV_H = float(1.0 / np.sqrt(np.float32(HID)))
INV_O = float(1.0 / np.sqrt(np.float32(OUT)))

_MESH = plsc.VectorSubcoreMesh(core_axis_name="c", subcore_axis_name="s")
_SC_PARAMS = pltpu.CompilerParams(
    needs_layout_passes=False, use_tc_tiling_on_sc=False)


def _worker():
    c = lax.axis_index("c")
    s = lax.axis_index("s")
    wid = s * 2 + c
    nch = jnp.where(wid < EXTRA, CPW + 1, CPW)
    base = wid * CPW + jnp.minimum(wid, EXTRA)
    return c, s, wid, nch, base


def _zero_acc(zeros, acc, stg, rows0):
    pltpu.sync_copy(zeros.at[pl.ds(0, STG)], stg)
    for i in range(TPR // STG):
        pltpu.sync_copy(stg, acc.at[pl.ds(rows0 + i * STG, STG)])
    plsc.subcore_barrier()


def _flush_acc(acc, accout, stg, cidx, rows0):
    plsc.subcore_barrier()
    for i in range(TPR // STG):
        pltpu.sync_copy(acc.at[pl.ds(rows0 + i * STG, STG)], stg)
        pltpu.sync_copy(stg, accout.at[cidx, pl.ds(rows0 + i * STG, STG)])


def _read_gmax(wmax, wbuf):
    pltpu.sync_copy(wmax, wbuf)
    iota = lax.iota(jnp.int32, 16)
    m = jnp.full((16,), -jnp.inf, jnp.float32)
    for i in range(NW):
        m = jnp.maximum(
            m, plsc.load_gather(wbuf, [jnp.full((16,), i, jnp.int32), iota]))
    return jnp.max(m)


def _preset_zero_cols(mbuf, lo, hi):
    iota = lax.iota(jnp.int32, 16)
    z = jnp.zeros((16,), jnp.float32)
    for g in range(CH // 16):
        rws = g * 16 + iota

        def cc(ci, carry):
            plsc.store_scatter(mbuf, [rws, jnp.full((16,), ci, jnp.int32)], z)
            return carry

        lax.fori_loop(lo, hi, cc, 0)


@functools.partial(
    pl.kernel,
    out_type=[
        jax.ShapeDtypeStruct((2, E), jnp.float32),
        jax.ShapeDtypeStruct((NW, 16), jnp.float32),
    ],
    mesh=_MESH,
    compiler_params=_SC_PARAMS,
    scratch_types=[
        pltpu.VMEM((CH,), jnp.int32),
        pltpu.VMEM((CH,), jnp.int32),
        pltpu.VMEM((CH, 2 * HID), jnp.float32),
        pltpu.VMEM((CH, 2 * HID), jnp.float32),
        pltpu.VMEM((CH,), jnp.float32),
        pltpu.VMEM((CH,), jnp.float32),
        pltpu.VMEM((16,), jnp.float32),
        pltpu.SemaphoreType.DMA,
        pltpu.SemaphoreType.DMA,
    ],
)
def _alpha1(qtab, ktab, edge, alpha, wmax, dbuf, sbuf, qbuf, kbuf, apb, asb,
            mst, sem1, sem2):
    _, _, wid, nch, base = _worker()
    iota = lax.iota(jnp.int32, 16)

    def chunk(j, wm):
        off = (base + j) * CH
        pltpu.sync_copy(edge.at[1, pl.ds(off, CH)], dbuf)
        pltpu.sync_copy(edge.at[0, pl.ds(off, CH)], sbuf)
        cq = pltpu.async_copy(qtab.at[dbuf], qbuf, sem1)
        ck = pltpu.async_copy(ktab.at[sbuf], kbuf, sem2)
        cq.wait()
        ck.wait()
        for g in range(CH // 16):
            rows = g * 16 + iota

            def cc(ci, acc):
                ap, as_ = acc
                colp = jnp.full((16,), ci, jnp.int32)
                cols = colp + HID
                qp = plsc.load_gather(qbuf, [rows, colp])
                kp = plsc.load_gather(kbuf, [rows, colp])
                qs = plsc.load_gather(qbuf, [rows, cols])
                ks = plsc.load_gather(kbuf, [rows, cols])
                return ap + qp * kp, as_ + qs * ks

            z = jnp.zeros((16,), jnp.float32)
            ap, as_ = lax.fori_loop(0, HID, cc, (z, z))
            ap = ap * INV_H
            as_ = as_ * INV_H
            apb[pl.ds(g * 16, 16)] = ap
            asb[pl.ds(g * 16, 16)] = as_
            wm = jnp.maximum(wm, jnp.maximum(ap, as_))
        pltpu.sync_copy(apb, alpha.at[0, pl.ds(off, CH)])
        pltpu.sync_copy(asb, alpha.at[1, pl.ds(off, CH)])
        return wm

    wm = lax.fori_loop(0, nch, chunk, jnp.full((16,), -jnp.inf, jnp.float32))
    mst[...] = wm
    pltpu.sync_copy(mst, wmax.at[wid])


@functools.partial(
    pl.kernel,
    out_type=jax.ShapeDtypeStruct((2, NP, 16), jnp.float32),
    mesh=_MESH,
    compiler_params=_SC_PARAMS,
    scratch_types=[
        pltpu.VMEM((CH,), jnp.int32),
        pltpu.VMEM((CH,), jnp.float32),
        pltpu.VMEM((CH,), jnp.float32),
        pltpu.VMEM((CH, 16), jnp.float32),
        pltpu.VMEM((NW, 16), jnp.float32),
        pltpu.VMEM((STG, 16), jnp.float32),
        pltpu.VMEM_SHARED((NP, 16), jnp.float32),
    ],
)
def _den1(alpha, wmax, edge, zeros, accout, dbuf, apb, asb, mbuf, wbuf, stg,
          acc):
    cidx, s, wid, nch, base = _worker()
    iota = lax.iota(jnp.int32, 16)
    gmax = _read_gmax(wmax, wbuf)
    rows0 = s * TPR
    _zero_acc(zeros, acc, stg, rows0)
    _preset_zero_cols(mbuf, 2, 16)

    def chunk(j, _):
        off = (base + j) * CH
        pltpu.sync_copy(edge.at[1, pl.ds(off, CH)], dbuf)
        pltpu.sync_copy(alpha.at[0, pl.ds(off, CH)], apb)
        pltpu.sync_copy(alpha.at[1, pl.ds(off, CH)], asb)
        for g in range(CH // 16):
            rws = g * 16 + iota
            ep = jnp.exp(apb[pl.ds(g * 16, 16)] - gmax)
            es = jnp.exp(asb[pl.ds(g * 16, 16)] - gmax)
            plsc.store_scatter(mbuf, [rws, jnp.full((16,), 0, jnp.int32)], ep)
            plsc.store_scatter(mbuf, [rws, jnp.full((16,), 1, jnp.int32)], es)
        pltpu.sync_copy(mbuf, acc.at[dbuf], add=True)
        return 0

    lax.fori_loop(0, nch, chunk, 0)
    _flush_acc(acc, accout, stg, cidx, rows0)


@functools.partial(
    pl.kernel,
    out_type=jax.ShapeDtypeStruct((2, NP, 16), jnp.float32),
    mesh=_MESH,
    compiler_params=_SC_PARAMS,
    scratch_types=[
        pltpu.VMEM((CH,), jnp.int32),
        pltpu.VMEM((CH,), jnp.int32),
        pltpu.VMEM((CH,), jnp.float32),
        pltpu.VMEM((CH, 16), jnp.float32),
        pltpu.VMEM((CH, 16), jnp.float32),
        pltpu.VMEM((NW, 16), jnp.float32),
        pltpu.VMEM((STG, 16), jnp.float32),
        pltpu.VMEM_SHARED((NP, 16), jnp.float32),
        pltpu.SemaphoreType.DMA,
    ],
)
def _chan16(vtab, alpha_b, wmax, edge, zeros, accout, dbuf, sbuf, abuf, vbuf,
            mbuf, wbuf, stg, acc, sem):
    cidx, s, wid, nch, base = _worker()
    iota = lax.iota(jnp.int32, 16)
    gmax = _read_gmax(wmax, wbuf)
    rows0 = s * TPR
    _zero_acc(zeros, acc, stg, rows0)

    def chunk(j, _):
        off = (base + j) * CH
        pltpu.sync_copy(edge.at[1, pl.ds(off, CH)], dbuf)
        pltpu.sync_copy(edge.at[0, pl.ds(off, CH)], sbuf)
        pltpu.sync_copy(alpha_b.at[pl.ds(off, CH)], abuf)
        pltpu.async_copy(vtab.at[sbuf], vbuf, sem).wait()
        for g in range(CH // 16):
            rws = g * 16 + iota
            e = jnp.exp(abuf[pl.ds(g * 16, 16)] - gmax)

            def cc(ci, carry):
                col = jnp.full((16,), ci, jnp.int32)
                v = plsc.load_gather(vbuf, [rws, col])
                plsc.store_scatter(mbuf, [rws, col], e * v)
                return carry

            lax.fori_loop(0, 16, cc, 0)
        pltpu.sync_copy(mbuf, acc.at[dbuf], add=True)
        return 0

    lax.fori_loop(0, nch, chunk, 0)
    _flush_acc(acc, accout, stg, cidx, rows0)


@functools.partial(
    pl.kernel,
    out_type=jax.ShapeDtypeStruct((2, NP, 16), jnp.float32),
    mesh=_MESH,
    compiler_params=_SC_PARAMS,
    scratch_types=[
        pltpu.VMEM((CH,), jnp.int32),
        pltpu.VMEM((CH,), jnp.int32),
        pltpu.VMEM((CH, OUT), jnp.float32),
        pltpu.VMEM((CH, OUT), jnp.float32),
        pltpu.VMEM((CH, 16), jnp.float32),
        pltpu.VMEM((CH, 16), jnp.float32),
        pltpu.VMEM((STG, 16), jnp.float32),
        pltpu.VMEM_SHARED((NP, 16), jnp.float32),
        pltpu.SemaphoreType.DMA,
        pltpu.SemaphoreType.DMA,
        pltpu.SemaphoreType.DMA,
    ],
)
def _layer2(qtab, ktab, vtab, edge, zeros, accout, dbuf, sbuf, qbuf, kbuf,
            vbuf, mbuf, stg, acc, sem1, sem2, sem3):
    cidx, s, wid, nch, base = _worker()
    iota = lax.iota(jnp.int32, 16)
    rows0 = s * TPR
    _zero_acc(zeros, acc, stg, rows0)
    _preset_zero_cols(mbuf, OUT + 1, 16)

    def chunk(j, _):
        off = (base + j) * CH
        pltpu.sync_copy(edge.at[1, pl.ds(off, CH)], dbuf)
        pltpu.sync_copy(edge.at[0, pl.ds(off, CH)], sbuf)
        cq = pltpu.async_copy(qtab.at[dbuf], qbuf, sem1)
        ck = pltpu.async_copy(ktab.at[sbuf], kbuf, sem2)
        cv = pltpu.async_copy(vtab.at[sbuf], vbuf, sem3)
        cq.wait()
        ck.wait()
        cv.wait()
        for g in range(CH // 16):
            rws = g * 16 + iota

            def cc(ci, a):
                col = jnp.full((16,), ci, jnp.int32)
                q = plsc.load_gather(qbuf, [rws, col])
                k = plsc.load_gather(kbuf, [rws, col])
                return a + q * k

            a = lax.fori_loop(0, OUT, cc, jnp.zeros((16,), jnp.float32))
            ex = jnp.exp(a * INV_O)

            def cv2(ci, carry):
                col = jnp.full((16,), ci, jnp.int32)
                v = plsc.load_gather(vbuf, [rws, col])
                plsc.store_scatter(mbuf, [rws, col], ex * v)
                return carry

            lax.fori_loop(0, OUT + 1, cv2, 0)
        pltpu.sync_copy(mbuf, acc.at[dbuf], add=True)
        return 0

    lax.fori_loop(0, nch, chunk, 0)
    _flush_acc(acc, accout, stg, cidx, rows0)


def _t0_body(xo_ref, bt_ref, xdep_ref, wa_ref, ba_ref, wb_ref, bb_ref,
             q1_ref, k1_ref, vpl_ref, vph_ref, vsl_ref, vsh_ref, sp_ref,
             ss_ref):
    xo = xo_ref[...]
    bt = bt_ref[...]
    cols = lax.broadcasted_iota(jnp.int32, (RB, B), 1)
    oh = bt == cols
    xd = jnp.sum(jnp.where(oh, xdep_ref[...], 0.0), axis=1, keepdims=True)
    pa = xo * wa_ref[0:1, :] + xd * wa_ref[1:2, :] + ba_ref[...]
    pb = xo * wb_ref[0:1, :] + xd * wb_ref[1:2, :] + bb_ref[...]
    q1_ref[...] = jnp.concatenate([pa[:, 0:32], pb[:, 0:32]], axis=1)
    k1_ref[...] = jnp.concatenate([pa[:, 32:64], pb[:, 32:64]], axis=1)
    vpl_ref[...] = pa[:, 64:80]
    vph_ref[...] = pa[:, 80:96]
    vsl_ref[...] = pb[:, 64:80]
    vsh_ref[...] = pb[:, 80:96]
    sp_ref[...] = pa[:, 96:128]
    ss_ref[...] = pb[:, 96:128]


def _t1_body(den_ref, apl_ref, aph_ref, asl_ref, ash_ref, sp_ref, ss_ref,
             wp_ref, bp_ref, ws_ref, bs_ref, qp_ref, kp_ref, vp_ref, qs_ref,
             ks_ref, vs_ref, s2p_ref, s2s_ref):
    onesz = jnp.concatenate(
        [jnp.ones((RB, 1), jnp.float32), jnp.zeros((RB, 7), jnp.float32)],
        axis=1)

    def combine(lo_ref, hi_ref, skip_ref, dcol):
        num = jnp.concatenate(
            [lo_ref[0] + lo_ref[1], hi_ref[0] + hi_ref[1]], axis=1)
        den = den_ref[0, :, dcol:dcol + 1] + den_ref[1, :, dcol:dcol + 1]
        return num / (den + 1e-16) + skip_ref[...]

    hp = combine(apl_ref, aph_ref, sp_ref, 0)
    hs = combine(asl_ref, ash_ref, ss_ref, 1)
    pp = jnp.dot(hp, wp_ref[...], preferred_element_type=jnp.float32) + bp_ref[...]
    ps = jnp.dot(hs, ws_ref[...], preferred_element_type=jnp.float32) + bs_ref[...]
    qp_ref[...] = pp[:, 0:8]
    kp_ref[...] = pp[:, 8:16]
    vp_ref[...] = jnp.concatenate([pp[:, 16:24], onesz], axis=1)
    qs_ref[...] = ps[:, 0:8]
    ks_ref[...] = ps[:, 8:16]
    vs_ref[...] = jnp.concatenate([ps[:, 16:24], onesz], axis=1)
    s2p_ref[...] = pp[:, 24:32]
    s2s_ref[...] = ps[:, 24:32]


def _t2_body(ap_ref, as_ref, sp_ref, ss_ref, op_ref, os_ref):
    def fin(a_ref, skip_ref):
        num = a_ref[0, :, 0:8] + a_ref[1, :, 0:8]
        den = a_ref[0, :, 8:9] + a_ref[1, :, 8:9]
        return jnp.tanh(num / (den + 1e-16) + skip_ref[...])

    op_ref[...] = fin(ap_ref, sp_ref)
    os_ref[...] = fin(as_ref, ss_ref)


def _rows(shape):
    nd = len(shape)
    if nd == 2:
        return pl.BlockSpec((RB, shape[1]), lambda i: (i, 0))
    return pl.BlockSpec((shape[0], RB, shape[2]), lambda i: (0, i, 0))


def _full(shape):
    return pl.BlockSpec(shape, lambda i: tuple(0 for _ in shape))


def _cat_w(p):
    w = jnp.concatenate([p["Wq"], p["Wk"], p["Wv"], p["Ws"]], axis=1)
    b = jnp.concatenate([p["bq"], p["bk"], p["bv"], p["bs"]]).reshape(1, -1)
    return w, b


def kernel(x_offset, x_depth, batch, edge_index, params):
    xo = x_offset.reshape(-1).astype(jnp.float32)
    xdep = x_depth.reshape(-1).astype(jnp.float32).reshape(1, B)
    bt = batch.reshape(-1).astype(jnp.int32)
    ei = edge_index.astype(jnp.int32)

    xo_p = jnp.pad(xo, (0, NP - N)).reshape(NP, 1)
    bt_p = jnp.pad(bt, (0, NP - N)).reshape(NP, 1)

    wa, ba = _cat_w(params["p1"])
    wb, bb = _cat_w(params["s1"])
    wp2, bp2 = _cat_w(params["p2"])
    ws2, bs2 = _cat_w(params["s2"])

    f32 = jnp.float32
    t0_outs = [
        jax.ShapeDtypeStruct((NP, 64), f32),
        jax.ShapeDtypeStruct((NP, 64), f32),
        jax.ShapeDtypeStruct((NP, 16), f32),
        jax.ShapeDtypeStruct((NP, 16), f32),
        jax.ShapeDtypeStruct((NP, 16), f32),
        jax.ShapeDtypeStruct((NP, 16), f32),
        jax.ShapeDtypeStruct((NP, 32), f32),
        jax.ShapeDtypeStruct((NP, 32), f32),
    ]
    q1, k1, vpl, vph, vsl, vsh, s1p, s1s = pl.pallas_call(
        _t0_body,
        grid=(NP // RB,),
        in_specs=[_rows((NP, 1)), _rows((NP, 1)), _full((1, B)),
                  _full((2, 128)), _full((1, 128)), _full((2, 128)),
                  _full((1, 128))],
        out_specs=[_rows(o.shape) for o in t0_outs],
        out_shape=t0_outs,
    )(xo_p, bt_p, xdep, wa, ba, wb, bb)

    alpha, wmax = _alpha1(q1, k1, ei)
    alpha_p = alpha[0]
    alpha_s = alpha[1]

    z16 = jnp.zeros((NP, 16), f32)
    den1 = _den1(alpha, wmax, ei, z16)
    apl = _chan16(vpl, alpha_p, wmax, ei, z16)
    aph = _chan16(vph, alpha_p, wmax, ei, z16)
    asl = _chan16(vsl, alpha_s, wmax, ei, z16)
    ash = _chan16(vsh, alpha_s, wmax, ei, z16)

    t1_outs = [
        jax.ShapeDtypeStruct((NP, 8), f32),
        jax.ShapeDtypeStruct((NP, 8), f32),
        jax.ShapeDtypeStruct((NP, 16), f32),
        jax.ShapeDtypeStruct((NP, 8), f32),
        jax.ShapeDtypeStruct((NP, 8), f32),
        jax.ShapeDtypeStruct((NP, 16), f32),
        jax.ShapeDtypeStruct((NP, 8), f32),
        jax.ShapeDtypeStruct((NP, 8), f32),
    ]
    q2p, k2p, v2p, q2s, k2s, v2s, s2p, s2s = pl.pallas_call(
        _t1_body,
        grid=(NP // RB,),
        in_specs=[_rows((2, NP, 16)), _rows((2, NP, 16)), _rows((2, NP, 16)),
                  _rows((2, NP, 16)), _rows((2, NP, 16)), _rows((NP, 32)),
                  _rows((NP, 32)), _full((32, 32)), _full((1, 32)),
                  _full((32, 32)), _full((1, 32))],
        out_specs=[_rows(o.shape) for o in t1_outs],
        out_shape=t1_outs,
    )(den1, apl, aph, asl, ash, s1p, s1s, wp2, bp2, ws2, bs2)

    a2p = _layer2(q2p, k2p, v2p, ei, z16)
    a2s = _layer2(q2s, k2s, v2s, ei, z16)

    t2_outs = [
        jax.ShapeDtypeStruct((NP, 8), f32),
        jax.ShapeDtypeStruct((NP, 8), f32),
    ]
    op, os_ = pl.pallas_call(
        _t2_body,
        grid=(NP // RB,),
        in_specs=[_rows((2, NP, 16)), _rows((2, NP, 16)), _rows((NP, 8)),
                  _rows((NP, 8))],
        out_specs=[_rows(o.shape) for o in t2_outs],
        out_shape=t2_outs,
    )(a2p, a2s, s2p, s2s)

    return op[:N], os_[:N]
